# baseline (speedup 1.0000x reference)
import functools

import jax
import jax.numpy as jnp
from jax import lax
from jax.experimental import pallas as pl
from jax.experimental.pallas import tpu as pltpu
from jax.experimental.pallas import tpu_sc as plsc

N_NODES = 10000
N_EDGES = 320000
D_FEAT = 128
K_HOPS = 2
LANES = 16

NUM_CORES = 2
NUM_SUBCORES = 16
NUM_WORKERS = NUM_CORES * NUM_SUBCORES
F_PER_TILE = D_FEAT // NUM_WORKERS

CHUNK = 8000
N_CHUNKS = N_EDGES // CHUNK
N_PAIRS = N_CHUNKS // 2
N_GROUPS = CHUNK // LANES
ROW = 2 * CHUNK

SLICE = F_PER_TILE * N_NODES
N_ZERO_FLAT = SLICE // LANES


def _gcn_body(x_t, ep_h, out, tab_a, tab_b, eb0, eb1, s0, s1):
    wid = lax.axis_index("s") * NUM_CORES + lax.axis_index("c")
    fbase = wid * SLICE

    pltpu.sync_copy(x_t.at[pl.ds(fbase, SLICE)], tab_a)

    zero16 = jnp.zeros((LANES,), jnp.float32)

    def zero_ref(ref):
        @plsc.parallel_loop(0, N_ZERO_FLAT, 1, unroll=8)
        def zbody(i):
            ref[pl.ds(i * LANES, LANES)] = zero16

    def compute(eb, table, acc):
        @plsc.parallel_loop(0, N_GROUPS, 1, unroll=8)
        def grp(i):
            gb = i * LANES
            sd16 = eb[pl.ds(gb, LANES)]
            s16 = sd16 & 0xFFFF
            d16 = lax.shift_right_logical(sd16, 16)
            w16 = plsc.bitcast(eb[pl.ds(CHUNK + gb, LANES)], jnp.float32)
            for f in range(F_PER_TILE):
                off = f * N_NODES
                g = plsc.load_gather(table, [s16 + off])
                plsc.addupdate_scatter(acc, [d16 + off], g * w16)

    def hop(table, acc):
        pltpu.async_copy(ep_h.at[0], eb0, s0)
        pltpu.async_copy(ep_h.at[1], eb1, s1)

        def pair(p, _):
            c = 2 * p
            pltpu.make_async_copy(ep_h.at[0], eb0, s0).wait()
            compute(eb0, table, acc)
            pltpu.async_copy(ep_h.at[c + 2], eb0, s0)
            pltpu.make_async_copy(ep_h.at[0], eb1, s1).wait()
            compute(eb1, table, acc)
            pltpu.async_copy(ep_h.at[c + 3], eb1, s1)
            return 0

        lax.fori_loop(0, N_PAIRS, pair, 0)
        pltpu.make_async_copy(ep_h.at[0], eb0, s0).wait()
        pltpu.make_async_copy(ep_h.at[0], eb1, s1).wait()

    zero_ref(tab_b)
    hop(tab_a, tab_b)
    zero_ref(tab_a)
    hop(tab_b, tab_a)

    pltpu.sync_copy(tab_a, out.at[pl.ds(fbase, SLICE)])


_gcn = functools.partial(
    pl.kernel,
    out_type=jax.ShapeDtypeStruct((D_FEAT * N_NODES,), jnp.float32),
    mesh=plsc.VectorSubcoreMesh(core_axis_name="c", subcore_axis_name="s"),
    compiler_params=pltpu.CompilerParams(needs_layout_passes=False),
    scratch_types=[
        pltpu.VMEM((SLICE,), jnp.float32),
        pltpu.VMEM((SLICE,), jnp.float32),
        pltpu.VMEM((ROW,), jnp.int32),
        pltpu.VMEM((ROW,), jnp.int32),
        pltpu.SemaphoreType.DMA,
        pltpu.SemaphoreType.DMA,
    ],
)(_gcn_body)


def kernel(x, edge_index, edge_weight):
    x_t = x.T.reshape(-1)
    wbits = lax.bitcast_convert_type(edge_weight, jnp.int32)
    sd = edge_index[0] | (edge_index[1] << 16)
    e2 = jnp.stack([sd, wbits])
    epack = (e2.reshape(2, N_CHUNKS, CHUNK)
             .transpose(1, 0, 2)
             .reshape(N_CHUNKS, ROW))
    epack = jnp.concatenate(
        [epack, jnp.zeros((2, ROW), jnp.int32)], axis=0)
    out_t = _gcn(x_t, epack)
    return out_t.reshape(D_FEAT, N_NODES).T
